# Initial kernel scaffold; baseline (speedup 1.0000x reference)
import jax
import jax.numpy as jnp
from jax.experimental import pallas as pl


def kernel(hidden_states, gate_w, w1, w3, w2):
    raise NotImplementedError("write your pallas kernel here")


# speedup vs baseline: 1.1260x; 1.1260x over previous
import functools

import jax
import jax.numpy as jnp
from jax import lax
from jax.experimental import pallas as pl
from jax.experimental.pallas import tpu as pltpu

_TOPK = 2
_TM = 128


def _router_body(x_ref, gw_ref, topi_ref, rw_ref):
    x = x_ref[...]
    gw = gw_ref[...]
    logits = lax.dot_general(x, gw, (((1,), (1,)), ((), ())),
                             preferred_element_type=jnp.float32)
    E = logits.shape[1]
    lane = lax.broadcasted_iota(jnp.int32, logits.shape, 1)
    m1 = jnp.max(logits, axis=1, keepdims=True)
    i1 = jnp.min(jnp.where(logits == m1, lane, E), axis=1,
                 keepdims=True)
    masked = jnp.where(lane == i1, -jnp.inf, logits)
    m2 = jnp.max(masked, axis=1, keepdims=True)
    i2 = jnp.min(jnp.where(masked == m2, lane, E), axis=1, keepdims=True)
    e2 = jnp.exp(m2 - m1)
    denom = 1.0 + e2
    r1 = 1.0 / denom
    r2 = e2 / denom
    topi_ref[...] = jnp.concatenate([i1, i2], axis=1).astype(jnp.int32)
    rw_ref[...] = jnp.concatenate([r1, r2], axis=1)


def _router(x, gate_w):
    T, _ = x.shape
    return pl.pallas_call(
        _router_body,
        out_shape=(
            jax.ShapeDtypeStruct((T, _TOPK), jnp.int32),
            jax.ShapeDtypeStruct((T, _TOPK), jnp.float32),
        ),
    )(x, gate_w)


def _dispatch_metadata(topi, rw, T, E):
    N = T * _TOPK
    ee = topi.reshape(-1).astype(jnp.int32)
    order = jnp.argsort(ee, stable=True)
    tok_sorted = (order // _TOPK).astype(jnp.int32)
    w_sorted = rw.reshape(-1)[order]

    counts = jnp.sum(ee[None, :] == jnp.arange(E)[:, None], axis=1)
    group_start = jnp.concatenate(
        [jnp.zeros((1,), jnp.int32), jnp.cumsum(counts)[:-1].astype(jnp.int32)])
    tpg = (counts + _TM - 1) // _TM
    tile_gstart = jnp.concatenate(
        [jnp.zeros((1,), jnp.int32), jnp.cumsum(tpg)[:-1].astype(jnp.int32)])
    total_tiles = jnp.sum(tpg).astype(jnp.int32)

    NT = N // _TM + E
    tid = jnp.arange(NT, dtype=jnp.int32)
    expert_id = jnp.clip(
        jnp.sum(tid[:, None] >= tile_gstart[None, :], axis=1) - 1, 0, E - 1
    ).astype(jnp.int32)
    valid_tile = (tid < total_tiles).astype(jnp.int32)
    local_tile = tid - tile_gstart[expert_id]
    row0 = group_start[expert_id] + local_tile * _TM
    nrows = jnp.clip(counts[expert_id] - local_tile * _TM, 0, _TM)

    j = jnp.arange(_TM, dtype=jnp.int32)
    s = row0[:, None] + j[None, :]
    vrow = j[None, :] < nrows[:, None]
    s_c = jnp.clip(s, 0, N - 1)
    tok_padded = jnp.where(vrow, tok_sorted[s_c], 0).astype(jnp.int32)
    w_padded = jnp.where(vrow, w_sorted[s_c], 0.0)

    p_flat = jnp.arange(NT * _TM, dtype=jnp.int32)
    s_flat = jnp.where(vrow.reshape(-1), s.reshape(-1), N)
    pos_of_sorted = jnp.zeros((N + 1,), jnp.int32).at[s_flat].set(p_flat)[:N]
    rank = jnp.zeros((N,), jnp.int32).at[order].set(
        jnp.arange(N, dtype=jnp.int32))
    pos_slots = pos_of_sorted[rank].reshape(T, _TOPK)
    return (NT, tok_padded.reshape(-1), w_padded, expert_id, valid_tile,
            pos_slots)


def _moe_tile_body(expert_sref, valid_sref, xs_ref, w1_ref, w3_ref, w2_ref,
                   wp_ref, y_ref):
    i = pl.program_id(0)

    @pl.when(valid_sref[i] == 1)
    def _():
        xt = xs_ref[...]
        w1e = w1_ref[0]
        w3e = w3_ref[0]
        w2e = w2_ref[0]
        g = lax.dot_general(xt, w1e, (((1,), (1,)), ((), ())),
                            preferred_element_type=jnp.float32)
        u = lax.dot_general(xt, w3e, (((1,), (1,)), ((), ())),
                            preferred_element_type=jnp.float32)
        h = (g * jax.nn.sigmoid(g)) * u
        y = lax.dot_general(h, w2e, (((1,), (1,)), ((), ())),
                            preferred_element_type=jnp.float32)
        y_ref[...] = y * wp_ref[0, 0][:, None]

    @pl.when(valid_sref[i] == 0)
    def _():
        y_ref[...] = jnp.zeros_like(y_ref)


def _grouped_mlp(xs, w1, w3, w2, w_padded, expert_id, valid_tile, NT):
    _, F, D = w1.shape
    grid_spec = pltpu.PrefetchScalarGridSpec(
        num_scalar_prefetch=2,
        grid=(NT,),
        in_specs=[
            pl.BlockSpec((_TM, D), lambda i, e, v: (i, 0)),
            pl.BlockSpec((1, F, D), lambda i, e, v: (e[i], 0, 0)),
            pl.BlockSpec((1, F, D), lambda i, e, v: (e[i], 0, 0)),
            pl.BlockSpec((1, D, F), lambda i, e, v: (e[i], 0, 0)),
            pl.BlockSpec((1, 1, _TM), lambda i, e, v: (i, 0, 0)),
        ],
        out_specs=pl.BlockSpec((_TM, D), lambda i, e, v: (i, 0)),
    )
    return pl.pallas_call(
        _moe_tile_body,
        grid_spec=grid_spec,
        out_shape=jax.ShapeDtypeStruct((NT * _TM, D), jnp.float32),
    )(expert_id, valid_tile, xs, w1, w3, w2,
      w_padded.reshape(NT, 1, _TM))


def kernel(hidden_states, gate_w, w1, w3, w2):
    orig_shape = hidden_states.shape
    D = orig_shape[-1]
    x = hidden_states.reshape(-1, D)
    T = x.shape[0]
    E = gate_w.shape[0]

    topi, rw = _router(x, gate_w)
    (NT, tok_padded, w_padded, expert_id, valid_tile,
     pos_slots) = _dispatch_metadata(topi, rw, T, E)

    xs = jnp.take(x, tok_padded, axis=0)
    y = _grouped_mlp(xs, w1, w3, w2, w_padded, expert_id, valid_tile, NT)
    out = (jnp.take(y, pos_slots[:, 0], axis=0)
           + jnp.take(y, pos_slots[:, 1], axis=0))
    return out.reshape(orig_shape)


# speedup vs baseline: 1.4738x; 1.3088x over previous
import jax
import jax.numpy as jnp
from jax import lax
from jax.experimental import pallas as pl
from jax.experimental.pallas import tpu as pltpu

_TOPK = 2
_TM = 128


def _router_dispatch_body(x_ref, gw_ref, rw_ref, pp_ref, eid_ref, valid_ref):
    x = x_ref[...]
    gw = gw_ref[...]
    logits = lax.dot_general(x, gw, (((1,), (1,)), ((), ())),
                             preferred_element_type=jnp.float32)
    T, E = logits.shape
    lane = lax.broadcasted_iota(jnp.int32, logits.shape, 1)
    m1 = jnp.max(logits, axis=1, keepdims=True)
    i1 = jnp.min(jnp.where(logits == m1, lane, E), axis=1,
                 keepdims=True)
    masked = jnp.where(lane == i1, -jnp.inf, logits)
    m2 = jnp.max(masked, axis=1, keepdims=True)
    i2 = jnp.min(jnp.where(masked == m2, lane, E), axis=1, keepdims=True)
    e2 = jnp.exp(m2 - m1)
    denom = 1.0 + e2
    rw_ref[...] = jnp.concatenate([1.0 / denom, e2 / denom], axis=1)

    oh1 = (lane == i1).astype(jnp.float32)
    oh2 = (lane == i2).astype(jnp.float32)
    m = oh1 + oh2
    nb = T // _TM
    slt = (lax.broadcasted_iota(jnp.int32, (_TM, _TM), 1) <
           lax.broadcasted_iota(jnp.int32, (_TM, _TM), 0)).astype(
               jnp.float32)
    off = jnp.zeros((1, E), jnp.float32)
    parts = []
    for b in range(nb):
        blk = m[b * _TM:(b + 1) * _TM, :]
        within = lax.dot_general(slt, blk, (((1,), (0,)), ((), ())),
                                 preferred_element_type=jnp.float32)
        parts.append(within + off)
        off = off + jnp.sum(blk, axis=0, keepdims=True)
    pref_excl = jnp.concatenate(parts, axis=0)
    rank1 = jnp.sum(oh1 * pref_excl, axis=1, keepdims=True)
    rank2 = jnp.sum(oh2 * pref_excl, axis=1, keepdims=True)
    cnt_row = off

    tiles_before = jnp.int32(0)
    tile_starts = []
    ps1 = jnp.zeros((T, 1), jnp.float32)
    ps2 = jnp.zeros((T, 1), jnp.float32)
    for e in range(E):
        cnt = cnt_row[0, e].astype(jnp.int32)
        tile_starts.append(tiles_before)
        start_rows = (tiles_before * _TM).astype(jnp.float32)
        ps1 = ps1 + oh1[:, e:e + 1] * start_rows
        ps2 = ps2 + oh2[:, e:e + 1] * start_rows
        tiles_before = tiles_before + (cnt + _TM - 1) // _TM
    pp1 = (ps1 + rank1).astype(jnp.int32)
    pp2 = (ps2 + rank2).astype(jnp.int32)
    pp_ref[...] = jnp.concatenate([pp1, pp2], axis=1)

    total_tiles = tiles_before
    tid = lax.broadcasted_iota(jnp.int32, (1, 128), 1)
    eid = jnp.zeros((1, 128), jnp.int32)
    for e in range(1, E):
        eid = eid + (tid >= tile_starts[e]).astype(jnp.int32)
    eid_ref[...] = eid
    valid_ref[...] = (tid < total_tiles).astype(jnp.int32)


def _router_dispatch(x, gate_w):
    T, _ = x.shape
    return pl.pallas_call(
        _router_dispatch_body,
        out_shape=(
            jax.ShapeDtypeStruct((T, _TOPK), jnp.float32),
            jax.ShapeDtypeStruct((T, _TOPK), jnp.int32),
            jax.ShapeDtypeStruct((1, 128), jnp.int32),
            jax.ShapeDtypeStruct((1, 128), jnp.int32),
        ),
    )(x, gate_w)


def _moe_tile_body(eid_sref, valid_sref, xs_ref, w1_ref, w3_ref, w2_ref,
                   y_ref):
    i = pl.program_id(0)

    @pl.when(valid_sref[0, i] == 1)
    def _():
        xt = xs_ref[...]
        w1e = w1_ref[0]
        w3e = w3_ref[0]
        w2e = w2_ref[0]
        g = lax.dot_general(xt, w1e, (((1,), (1,)), ((), ())),
                            preferred_element_type=jnp.float32)
        u = lax.dot_general(xt, w3e, (((1,), (1,)), ((), ())),
                            preferred_element_type=jnp.float32)
        h = (g * jax.nn.sigmoid(g)) * u
        y_ref[...] = lax.dot_general(h, w2e, (((1,), (1,)), ((), ())),
                                     preferred_element_type=jnp.float32)


def _grouped_mlp(xs, w1, w3, w2, eid, valid, NT):
    _, F, D = w1.shape
    grid_spec = pltpu.PrefetchScalarGridSpec(
        num_scalar_prefetch=2,
        grid=(NT,),
        in_specs=[
            pl.BlockSpec((_TM, D), lambda i, e, v: (i, 0)),
            pl.BlockSpec((1, F, D), lambda i, e, v: (e[0, i], 0, 0)),
            pl.BlockSpec((1, F, D), lambda i, e, v: (e[0, i], 0, 0)),
            pl.BlockSpec((1, D, F), lambda i, e, v: (e[0, i], 0, 0)),
        ],
        out_specs=pl.BlockSpec((_TM, D), lambda i, e, v: (i, 0)),
    )
    return pl.pallas_call(
        _moe_tile_body,
        grid_spec=grid_spec,
        out_shape=jax.ShapeDtypeStruct((NT * _TM, D), jnp.float32),
    )(eid, valid, xs, w1, w3, w2)


def kernel(hidden_states, gate_w, w1, w3, w2):
    orig_shape = hidden_states.shape
    D = orig_shape[-1]
    x = hidden_states.reshape(-1, D)
    T = x.shape[0]
    E = gate_w.shape[0]
    N = T * _TOPK
    NT = N // _TM + E

    rw, pp, eid, valid = _router_dispatch(x, gate_w)
    pad_pos = pp.reshape(-1)
    tok = jnp.arange(N, dtype=jnp.int32) // _TOPK
    tok_padded = jnp.zeros((NT * _TM,), jnp.int32).at[pad_pos].set(tok)

    xs = jnp.take(x, tok_padded, axis=0)
    y = _grouped_mlp(xs, w1, w3, w2, eid, valid, NT)

    out = (jnp.take(y, pp[:, 0], axis=0) * rw[:, 0:1]
           + jnp.take(y, pp[:, 1], axis=0) * rw[:, 1:2])
    return out.reshape(orig_shape)
